# baseline (speedup 1.0000x reference)
import functools

import jax
import jax.numpy as jnp
from jax import lax
from jax.experimental import pallas as pl
from jax.experimental.pallas import tpu as pltpu
from jax.experimental.pallas import tpu_sc as plsc

B, N, F, E = 4, 10000, 128, 320000

NC, NS, LANES = 2, 16, 16
NTILES = NC * NS
EDGES_PER_TILE = E // NTILES
K = 400
GROUPS = K // LANES
CHUNKS = EDGES_PER_TILE // K
ACC_W = 8
NPAD = 10240
ACC_LEN = ACC_W * NPAD


@functools.cache
def _make_sc_edge_scatter():
    sc_mesh = plsc.VectorSubcoreMesh(
        core_axis_name="c", subcore_axis_name="s", num_cores=NC, num_subcores=NS
    )
    return pl.kernel(
        _sc_edge_scatter_body,
        out_type=jax.ShapeDtypeStruct((NTILES * ACC_LEN,), jnp.float32),
        mesh=sc_mesh,
        compiler_params=pltpu.CompilerParams(
            use_tc_tiling_on_sc=False, needs_layout_passes=False),
        scratch_types=[
            pltpu.VMEM((B * N,), jnp.float32),
            pltpu.VMEM((ACC_LEN,), jnp.float32),
            pltpu.VMEM((K,), jnp.int32),
            pltpu.VMEM((K,), jnp.int32),
            pltpu.VMEM((B, K), jnp.float32),
            pltpu.VMEM((B, K), jnp.float32),
            pltpu.VMEM((B, K), jnp.float32),
            pltpu.SemaphoreType.DMA,
            pltpu.SemaphoreType.DMA,
        ],
    )


def _sc_edge_scatter_body(volt_hbm, src_hbm, dst_hbm, lf_hbm, tl_hbm, sus_hbm,
                          acc_hbm, volt_v, acc_v, src_v, dst_v, lf_v, tl_v,
                          sus_v, sem_in, sem_v):
    core = lax.axis_index("c")
    sub = lax.axis_index("s")

    cp_volt = pltpu.async_copy(volt_hbm, volt_v, sem_v)

    z16 = jnp.zeros((LANES,), jnp.float32)

    @pl.loop(0, ACC_LEN // LANES)
    def _(i):
        acc_v[pl.ds(i * LANES, LANES)] = z16

    cp_volt.wait()

    ebase = (core * NS + sub) * EDGES_PER_TILE

    @pl.loop(0, CHUNKS)
    def _(c):
        e0 = ebase + c * K
        cps = [
            pltpu.async_copy(src_hbm.at[pl.ds(e0, K)], src_v, sem_in),
            pltpu.async_copy(dst_hbm.at[pl.ds(e0, K)], dst_v, sem_in),
        ]
        for b in range(B):
            cps.append(pltpu.async_copy(
                lf_hbm.at[pl.ds(b * E + e0, K)], lf_v.at[b], sem_in))
            cps.append(pltpu.async_copy(
                tl_hbm.at[pl.ds(b * E + e0, K)], tl_v.at[b], sem_in))
            cps.append(pltpu.async_copy(
                sus_hbm.at[pl.ds(b * E + e0, K)], sus_v.at[b], sem_in))
        for cp in cps:
            cp.wait()

        for g in range(GROUPS):
            s16 = src_v[pl.ds(g * LANES, LANES)]
            d16 = dst_v[pl.ds(g * LANES, LANES)]
            for b in range(B):
                sb = s16 + (b * N) if b else s16
                db = d16 + (b * N) if b else d16
                vi = plsc.load_gather(volt_v, [sb])
                vj = plsc.load_gather(volt_v, [db])
                sus16 = sus_v[b, pl.ds(g * LANES, LANES)]
                fij = sus16 * vj / (vi + 1e-6)
                sa = s16 + (b * NPAD) if b else s16
                plsc.addupdate_scatter(acc_v, [sa], fij)
                lf16 = lf_v[b, pl.ds(g * LANES, LANES)]
                tl16 = tl_v[b, pl.ds(g * LANES, LANES)]
                loading = lf16 / (tl16 + 1e-6)
                ind = jnp.where(loading > 1.0, 1.0, 0.0).astype(jnp.float32)
                plsc.addupdate_scatter(acc_v, [sa + (4 * NPAD)], ind)

    tile = core * NS + sub
    pltpu.sync_copy(acc_v, acc_hbm.at[pl.ds(tile * ACC_LEN, ACC_LEN)])


BN = 2048


def _tc_body(nf_ref, acc_ref, out_ref):
    a = acc_ref[0, 0]
    for t in range(1, NTILES):
        a = a + acc_ref[0, t]
    out_ref[:, :, 0:F] = nf_ref[...]
    for b in range(B):
        out_ref[b, :, F] = (a[B + b] > 0.0).astype(jnp.float32)
        out_ref[b, :, F + 1] = jnp.abs(1.0 - a[b])


_tc_finalize = pl.pallas_call(
    _tc_body,
    grid=(NPAD // BN,),
    in_specs=[
        pl.BlockSpec((B, BN, F), lambda i: (0, i, 0)),
        pl.BlockSpec((1, NTILES, ACC_W, BN), lambda i: (0, 0, 0, i)),
    ],
    out_specs=pl.BlockSpec((B, BN, F + 2), lambda i: (0, i, 0)),
    out_shape=jax.ShapeDtypeStruct((B, NPAD, F + 2), jnp.float32),
)


def kernel(node_features, edge_index, line_flows, thermal_limits, susceptance):
    volt = node_features[:, :, 0].reshape(B * N)
    src = edge_index[0]
    dst = edge_index[1]
    lf1 = line_flows[:, :, 0].reshape(B * E)
    tl1 = thermal_limits[:, :, 0].reshape(B * E)
    sus1 = susceptance[:, :, 0].reshape(B * E)
    acc = _make_sc_edge_scatter()(volt, src, dst, lf1, tl1, sus1)
    out = _tc_finalize(node_features, acc.reshape(1, NTILES, ACC_W, NPAD))
    return out[:, :N, :]


# speedup vs baseline: 2.5599x; 2.5599x over previous
import functools

import jax
import jax.numpy as jnp
from jax import lax
from jax.experimental import pallas as pl
from jax.experimental.pallas import tpu as pltpu
from jax.experimental.pallas import tpu_sc as plsc

B, N, F, E = 4, 10000, 128, 320000

NC, NS, LANES = 2, 16, 16
NTILES = NC * NS
EDGES_PER_TILE = E // NTILES
K = 400
GROUPS = K // LANES
CHUNKS = EDGES_PER_TILE // K
ACC_W = 8
NPAD = 10240
ACC_LEN = ACC_W * NPAD


@functools.cache
def _make_sc_edge_scatter():
    sc_mesh = plsc.VectorSubcoreMesh(
        core_axis_name="c", subcore_axis_name="s", num_cores=NC, num_subcores=NS
    )
    return pl.kernel(
        _sc_edge_scatter_body,
        out_type=jax.ShapeDtypeStruct((NTILES * ACC_LEN,), jnp.float32),
        mesh=sc_mesh,
        compiler_params=pltpu.CompilerParams(
            use_tc_tiling_on_sc=False, needs_layout_passes=False),
        scratch_types=(
            [pltpu.VMEM((N,), jnp.float32)] * B
            + [pltpu.VMEM((NPAD,), jnp.float32)] * B
            + [pltpu.VMEM((NPAD,), jnp.float32)] * B
            + [
                pltpu.VMEM((K,), jnp.int32),
                pltpu.VMEM((K,), jnp.int32),
                pltpu.VMEM((B, K), jnp.float32),
                pltpu.VMEM((B, K), jnp.float32),
                pltpu.VMEM((B, K), jnp.float32),
                pltpu.SemaphoreType.DMA,
                pltpu.SemaphoreType.DMA,
            ]
        ),
    )


def _sc_edge_scatter_body(volt_hbm, src_hbm, dst_hbm, lf_hbm, tl_hbm, sus_hbm,
                          acc_hbm, v0, v1, v2, v3, f0, f1, f2, f3,
                          o0, o1, o2, o3, src_v, dst_v, lf_v, tl_v,
                          sus_v, sem_in, sem_v):
    core = lax.axis_index("c")
    sub = lax.axis_index("s")
    volt = [v0, v1, v2, v3]
    facc = [f0, f1, f2, f3]
    oacc = [o0, o1, o2, o3]

    cps_v = [pltpu.async_copy(volt_hbm.at[pl.ds(b * N, N)], volt[b], sem_v)
             for b in range(B)]

    z16 = jnp.zeros((LANES,), jnp.float32)
    one16 = jnp.full((LANES,), 1.0, jnp.float32)

    @plsc.parallel_loop(0, NPAD // LANES)
    def _(i):
        for b in range(B):
            facc[b][pl.ds(i * LANES, LANES)] = z16
            oacc[b][pl.ds(i * LANES, LANES)] = z16

    for cp in cps_v:
        cp.wait()

    ebase = (core * NS + sub) * EDGES_PER_TILE

    @pl.loop(0, CHUNKS)
    def _(c):
        e0 = ebase + c * K
        cps = [
            pltpu.async_copy(src_hbm.at[pl.ds(e0, K)], src_v, sem_in),
            pltpu.async_copy(dst_hbm.at[pl.ds(e0, K)], dst_v, sem_in),
        ]
        for b in range(B):
            cps.append(pltpu.async_copy(
                lf_hbm.at[pl.ds(b * E + e0, K)], lf_v.at[b], sem_in))
            cps.append(pltpu.async_copy(
                tl_hbm.at[pl.ds(b * E + e0, K)], tl_v.at[b], sem_in))
            cps.append(pltpu.async_copy(
                sus_hbm.at[pl.ds(b * E + e0, K)], sus_v.at[b], sem_in))
        for cp in cps:
            cp.wait()

        @plsc.parallel_loop(0, GROUPS, unroll=2)
        def _(g):
            s16 = src_v[pl.ds(g * LANES, LANES)]
            d16 = dst_v[pl.ds(g * LANES, LANES)]
            for b in range(B):
                vi = plsc.load_gather(volt[b], [s16])
                vj = plsc.load_gather(volt[b], [d16])
                sus16 = sus_v[b, pl.ds(g * LANES, LANES)]
                fij = sus16 * vj / (vi + 1e-6)
                plsc.addupdate_scatter(facc[b], [s16], fij)
                lf16 = lf_v[b, pl.ds(g * LANES, LANES)]
                tl16 = tl_v[b, pl.ds(g * LANES, LANES)]
                plsc.store_scatter(oacc[b], [s16], one16,
                                   mask=lf16 > tl16 + 1e-6)

    tile = core * NS + sub
    for b in range(B):
        pltpu.sync_copy(
            facc[b], acc_hbm.at[pl.ds(tile * ACC_LEN + b * NPAD, NPAD)])
        pltpu.sync_copy(
            oacc[b], acc_hbm.at[pl.ds(tile * ACC_LEN + (B + b) * NPAD, NPAD)])


BN = 2048


def _tc_body(nf_ref, acc_ref, out_ref):
    a = acc_ref[0, 0]
    for t in range(1, NTILES):
        a = a + acc_ref[0, t]
    out_ref[:, :, 0:F] = nf_ref[...]
    for b in range(B):
        out_ref[b, :, F] = (a[B + b] > 0.0).astype(jnp.float32)
        out_ref[b, :, F + 1] = jnp.abs(1.0 - a[b])


_tc_finalize = pl.pallas_call(
    _tc_body,
    grid=(NPAD // BN,),
    in_specs=[
        pl.BlockSpec((B, BN, F), lambda i: (0, i, 0)),
        pl.BlockSpec((1, NTILES, ACC_W, BN), lambda i: (0, 0, 0, i)),
    ],
    out_specs=pl.BlockSpec((B, BN, F + 2), lambda i: (0, i, 0)),
    out_shape=jax.ShapeDtypeStruct((B, N, F + 2), jnp.float32),
)


def kernel(node_features, edge_index, line_flows, thermal_limits, susceptance):
    volt = node_features[:, :, 0].reshape(B * N)
    src = edge_index[0]
    dst = edge_index[1]
    lf1 = line_flows[:, :, 0].reshape(B * E)
    tl1 = thermal_limits[:, :, 0].reshape(B * E)
    sus1 = susceptance[:, :, 0].reshape(B * E)
    acc = _make_sc_edge_scatter()(volt, src, dst, lf1, tl1, sus1)
    return _tc_finalize(node_features, acc.reshape(1, NTILES, ACC_W, NPAD))


# speedup vs baseline: 3.1727x; 1.2394x over previous
import functools

import jax
import jax.numpy as jnp
from jax import lax
from jax.experimental import pallas as pl
from jax.experimental.pallas import tpu as pltpu
from jax.experimental.pallas import tpu_sc as plsc

B, N, F, E = 4, 10000, 128, 320000

NC, NS, LANES = 2, 16, 16
NTILES = NC * NS
EDGES_PER_TILE = E // NTILES
K = 400
GROUPS = K // LANES
CHUNKS = EDGES_PER_TILE // K
ACC_W = 8
NPAD = 10240
ACC_LEN = ACC_W * NPAD


@functools.cache
def _make_sc_edge_scatter():
    sc_mesh = plsc.VectorSubcoreMesh(
        core_axis_name="c", subcore_axis_name="s", num_cores=NC, num_subcores=NS
    )
    return pl.kernel(
        _sc_edge_scatter_body,
        out_type=jax.ShapeDtypeStruct((NTILES * ACC_LEN,), jnp.float32),
        mesh=sc_mesh,
        compiler_params=pltpu.CompilerParams(
            use_tc_tiling_on_sc=False, needs_layout_passes=False),
        scratch_types=(
            [pltpu.VMEM((N,), jnp.float32)] * B
            + [pltpu.VMEM((NPAD,), jnp.float32)] * B
            + [pltpu.VMEM((NPAD,), jnp.float32)] * B
            + [
                pltpu.VMEM((K,), jnp.int32),
                pltpu.VMEM((K,), jnp.int32),
                pltpu.VMEM((B, K), jnp.float32),
                pltpu.VMEM((B, K), jnp.float32),
                pltpu.VMEM((B, K), jnp.float32),
                pltpu.SemaphoreType.DMA,
                pltpu.SemaphoreType.DMA,
            ]
        ),
    )


def _sc_edge_scatter_body(volt_hbm, src_hbm, dst_hbm, lf_hbm, tl_hbm, sus_hbm,
                          acc_hbm, v0, v1, v2, v3, f0, f1, f2, f3,
                          o0, o1, o2, o3, src_v, dst_v, lf_v, tl_v,
                          sus_v, sem_in, sem_v):
    core = lax.axis_index("c")
    sub = lax.axis_index("s")
    volt = [v0, v1, v2, v3]
    facc = [f0, f1, f2, f3]
    oacc = [o0, o1, o2, o3]

    cps_v = [pltpu.async_copy(volt_hbm.at[pl.ds(b * N, N)], volt[b], sem_v)
             for b in range(B)]

    z16 = jnp.zeros((LANES,), jnp.float32)
    one16 = jnp.full((LANES,), 1.0, jnp.float32)

    @plsc.parallel_loop(0, NPAD // LANES)
    def _(i):
        for b in range(B):
            facc[b][pl.ds(i * LANES, LANES)] = z16
            oacc[b][pl.ds(i * LANES, LANES)] = z16

    for cp in cps_v:
        cp.wait()

    ebase = (core * NS + sub) * EDGES_PER_TILE

    @pl.loop(0, CHUNKS)
    def _(c):
        e0 = ebase + c * K
        cps = [
            pltpu.async_copy(src_hbm.at[pl.ds(e0, K)], src_v, sem_in),
            pltpu.async_copy(dst_hbm.at[pl.ds(e0, K)], dst_v, sem_in),
        ]
        for b in range(B):
            cps.append(pltpu.async_copy(
                lf_hbm.at[pl.ds(b * E + e0, K)], lf_v.at[b], sem_in))
            cps.append(pltpu.async_copy(
                tl_hbm.at[pl.ds(b * E + e0, K)], tl_v.at[b], sem_in))
            cps.append(pltpu.async_copy(
                sus_hbm.at[pl.ds(b * E + e0, K)], sus_v.at[b], sem_in))
        for cp in cps:
            cp.wait()

        @plsc.parallel_loop(0, GROUPS, unroll=2)
        def _(g):
            s16 = src_v[pl.ds(g * LANES, LANES)]
            d16 = dst_v[pl.ds(g * LANES, LANES)]
            for b in range(B):
                vi = plsc.load_gather(volt[b], [s16])
                vj = plsc.load_gather(volt[b], [d16])
                sus16 = sus_v[b, pl.ds(g * LANES, LANES)]
                fij = sus16 * vj / (vi + 1e-6)
                plsc.addupdate_scatter(facc[b], [s16], fij)
                lf16 = lf_v[b, pl.ds(g * LANES, LANES)]
                tl16 = tl_v[b, pl.ds(g * LANES, LANES)]
                plsc.store_scatter(oacc[b], [s16], one16,
                                   mask=lf16 > tl16 + 1e-6)

    tile = core * NS + sub
    for b in range(B):
        pltpu.sync_copy(
            facc[b], acc_hbm.at[pl.ds(tile * ACC_LEN + b * NPAD, NPAD)])
        pltpu.sync_copy(
            oacc[b], acc_hbm.at[pl.ds(tile * ACC_LEN + (B + b) * NPAD, NPAD)])


BN = 2048


def _tc_body(acc_ref, n1_ref, li_ref):
    a = acc_ref[0, 0]
    for t in range(1, NTILES):
        a = a + acc_ref[0, t]
    for b in range(B):
        n1_ref[b, :] = (a[B + b] > 0.0).astype(jnp.float32)
        li_ref[b, :] = jnp.abs(1.0 - a[b])


_tc_finalize = pl.pallas_call(
    _tc_body,
    grid=(NPAD // BN,),
    in_specs=[
        pl.BlockSpec((1, NTILES, ACC_W, BN), lambda i: (0, 0, 0, i)),
    ],
    out_specs=[
        pl.BlockSpec((B, BN), lambda i: (0, i)),
        pl.BlockSpec((B, BN), lambda i: (0, i)),
    ],
    out_shape=[
        jax.ShapeDtypeStruct((B, N), jnp.float32),
        jax.ShapeDtypeStruct((B, N), jnp.float32),
    ],
)


def kernel(node_features, edge_index, line_flows, thermal_limits, susceptance):
    volt = node_features[:, :, 0].reshape(B * N)
    src = edge_index[0]
    dst = edge_index[1]
    lf1 = line_flows[:, :, 0].reshape(B * E)
    tl1 = thermal_limits[:, :, 0].reshape(B * E)
    sus1 = susceptance[:, :, 0].reshape(B * E)
    acc = _make_sc_edge_scatter()(volt, src, dst, lf1, tl1, sus1)
    n1, li = _tc_finalize(acc.reshape(1, NTILES, ACC_W, NPAD))
    return jnp.concatenate(
        [node_features, n1[:, :, None], li[:, :, None]], axis=2)
